# baseline (speedup 1.0000x reference)
import functools

import jax
import jax.numpy as jnp
from jax import lax
from jax.experimental import pallas as pl
from jax.experimental.pallas import tpu as pltpu
from jax.experimental.pallas import tpu_sc as plsc

_MAX_COUNT = 255
_D = 128
_NC = 2
_NS = 16
_NW = _NC * _NS
_NB = 4


@functools.lru_cache(maxsize=None)
def _make_lookup(nrows2d):
    rows_per_w = nrows2d // _NW
    steps = rows_per_w
    assert steps % _NB == 0

    mesh = plsc.VectorSubcoreMesh(core_axis_name="c", subcore_axis_name="s")

    @functools.partial(
        pl.kernel,
        mesh=mesh,
        out_type=jax.ShapeDtypeStruct((nrows2d * _D, _D), jnp.float32),
        scratch_types=(
            [pltpu.VMEM_SHARED((_MAX_COUNT + 1, _D), jnp.float32)]
            + [pltpu.VMEM((_MAX_COUNT + 1, _D), jnp.float32)]
            + [pltpu.VMEM((1, 128), jnp.int32)] * _NB
            + [pltpu.VMEM((128, _D), jnp.float32)] * _NB
            + [pltpu.SemaphoreType.DMA] * (3 * _NB)
        ),
    )
    def lookup(idx_hbm, tbl_hbm, ext_hbm, out_hbm, tbl_sh, stage_v, *rest):
        idx_v = rest[0:_NB]
        rows_v = rest[_NB:2 * _NB]
        isem = rest[2 * _NB:3 * _NB]
        gsem = rest[3 * _NB:4 * _NB]
        wsem = rest[4 * _NB:5 * _NB]

        cid = lax.axis_index("c")
        sid = lax.axis_index("s")
        wid = sid * _NC + cid
        row0 = wid * rows_per_w

        @pl.when(sid == 0)
        def _():
            pltpu.sync_copy(tbl_hbm, stage_v)
            pltpu.sync_copy(ext_hbm, stage_v.at[pl.ds(_MAX_COUNT, 1)])
            pltpu.sync_copy(stage_v, tbl_sh)

        plsc.subcore_barrier()

        def fire_i(s, b):
            pltpu.async_copy(idx_hbm.at[pl.ds(row0 + s, 1)],
                             idx_v[b], isem[b])

        def wait_i(b):
            pltpu.make_async_copy(idx_hbm.at[pl.ds(row0, 1)],
                                  idx_v[b], isem[b]).wait()

        def fire_g(b):
            pltpu.async_copy(tbl_sh.at[idx_v[b].at[0]], rows_v[b], gsem[b])

        def wait_g(b):
            pltpu.make_async_copy(tbl_sh.at[idx_v[b].at[0]],
                                  rows_v[b], gsem[b]).wait()

        def fire_w(s, b):
            pltpu.async_copy(rows_v[b],
                             out_hbm.at[pl.ds((row0 + s) * 128, 128)],
                             wsem[b])

        def wait_w(b):
            pltpu.make_async_copy(rows_v[b],
                                  out_hbm.at[pl.ds(row0 * 128, 128)],
                                  wsem[b]).wait()

        fire_i(0, 0)

        def body(p, carry):
            for b in range(_NB):
                s = _NB * p + b
                wait_i(b)

                @pl.when(p >= 1)
                def _():
                    wait_w(b)

                fire_g(b)

                prev = (b - 1) % _NB
                if b == 0:
                    @pl.when(p >= 1)
                    def _():
                        wait_g(prev)
                        fire_w(s - 1, prev)
                else:
                    wait_g(prev)
                    fire_w(s - 1, prev)

                nxt = (b + 1) % _NB
                if b == _NB - 1:
                    @pl.when(p + 1 < steps // _NB)
                    def _():
                        fire_i(s + 1, nxt)
                else:
                    fire_i(s + 1, nxt)
            return carry

        lax.fori_loop(0, steps // _NB, body, 0)

        wait_g(_NB - 1)
        fire_w(steps - 1, _NB - 1)
        for b in range(_NB):
            wait_w(b)

    return lookup


def kernel(tensor, int_to_feat_matrix, extra_embeddings):
    batch, fields = tensor.shape
    total = batch * fields
    nrows2d = total // 128
    assert total % 128 == 0

    idx2d = tensor.astype(jnp.int32).reshape(nrows2d, 128)
    out2d = _make_lookup(nrows2d)(idx2d, int_to_feat_matrix, extra_embeddings)
    return out2d.reshape(batch, fields * _D)
